# baseline (speedup 1.0000x reference)
import jax
import jax.numpy as jnp
from jax import lax
from jax.experimental import pallas as pl
from jax.experimental.pallas import tpu as pltpu
from jax.experimental.pallas import tpu_sc as plsc

TOKENS = 32
VOCAB = 1000000
EPS = 1e-10
CHUNK = 65536
TCGRID = 15
TCV = TCGRID * CHUNK
TAIL = VOCAB - TCV
NSTRIP = TAIL // 16


def _tc_kernel(x_ref, e_ref, t_ref, o_ref, mx_ref, m_ref):
    i = pl.program_id(0)

    @pl.when(i == 0)
    def _init():
        m_ref[...] = jnp.full((TOKENS, 1), -jnp.inf, jnp.float32)
        o_ref[...] = jnp.zeros((TOKENS, 1), jnp.int32)

    x = x_ref[...]
    e = e_ref[...]
    t = t_ref[...]

    noise = jnp.log(e + EPS)
    key = x - t * noise

    idx = jax.lax.broadcasted_iota(jnp.int32, key.shape, 1)
    loc_max = jnp.max(key, axis=1, keepdims=True)
    hit = key == loc_max
    loc_arg = jnp.min(jnp.where(hit, idx, CHUNK), axis=1, keepdims=True)
    loc_arg = loc_arg + i * CHUNK

    better = loc_max > m_ref[...]
    m_ref[...] = jnp.where(better, loc_max, m_ref[...])
    o_ref[...] = jnp.where(better, loc_arg, o_ref[...])
    mx_ref[...] = m_ref[...]


def _noise_kernel(e_ref, n_ref):
    n_ref[...] = jnp.log(e_ref[...] + EPS)


def _sc_tail(x_hbm, n_hbm, t_hbm, mx_hbm, ix_hbm, xv, nv, tv, mxv, ixv):
    c = lax.axis_index("c")
    s = lax.axis_index("s")
    w = c * 16 + s

    pltpu.sync_copy(x_hbm.at[w], xv)
    pltpu.sync_copy(n_hbm.at[0], nv)
    pltpu.sync_copy(t_hbm.at[w], tv)
    il = lax.broadcasted_iota(jnp.int32, (16,), 0)
    t = tv[pl.ds(0, 16)]

    def body(g, carry):
        m, bi = carry
        for u in range(4):
            j = g * 4 + u
            xk = xv[pl.ds(j * 16, 16)]
            nk = nv[pl.ds(j * 16, 16)]
            key = xk - t * nk
            upd = key > m
            m = jnp.where(upd, key, m)
            bi = jnp.where(upd, il * 0 + j, bi)
        return m, bi

    m0 = jnp.full((16,), -jnp.inf, jnp.float32)
    b0 = jnp.zeros((16,), jnp.int32)
    m, bi = lax.fori_loop(0, NSTRIP // 4, body, (m0, b0))

    mxv[...] = m
    ixv[...] = bi * 16 + il + TCV
    pltpu.sync_copy(mxv, mx_hbm.at[w])
    pltpu.sync_copy(ixv, ix_hbm.at[w])


@jax.jit
def kernel(logits, temperatures, exponential):
    t = temperatures[:, None].astype(jnp.float32)

    x_tail = lax.slice(logits, (0, TCV), (TOKENS, VOCAB))
    e_tail = lax.slice(exponential, (0, TCV), (1, VOCAB))

    n_tail = pl.pallas_call(
        _noise_kernel,
        out_shape=jax.ShapeDtypeStruct((1, TAIL), jnp.float32),
    )(e_tail)

    sc = pl.kernel(
        _sc_tail,
        out_type=[
            jax.ShapeDtypeStruct((TOKENS, 16), jnp.float32),
            jax.ShapeDtypeStruct((TOKENS, 16), jnp.int32),
        ],
        scratch_types=[
            pltpu.VMEM((TAIL,), jnp.float32),
            pltpu.VMEM((TAIL,), jnp.float32),
            pltpu.VMEM((16,), jnp.float32),
            pltpu.VMEM((16,), jnp.float32),
            pltpu.VMEM((16,), jnp.int32),
        ],
        mesh=plsc.VectorSubcoreMesh(core_axis_name="c", subcore_axis_name="s"),
    )
    tb = jnp.broadcast_to(temperatures.astype(jnp.float32)[:, None], (TOKENS, 16))
    sc_max, sc_idx = sc(x_tail, n_tail, tb)

    tc_arg, tc_max = pl.pallas_call(
        _tc_kernel,
        grid=(TCGRID,),
        in_specs=[
            pl.BlockSpec((TOKENS, CHUNK), lambda i: (0, i)),
            pl.BlockSpec((1, CHUNK), lambda i: (0, i)),
            pl.BlockSpec((TOKENS, 1), lambda i: (0, 0)),
        ],
        out_specs=[
            pl.BlockSpec((TOKENS, 1), lambda i: (0, 0)),
            pl.BlockSpec((TOKENS, 1), lambda i: (0, 0)),
        ],
        out_shape=[
            jax.ShapeDtypeStruct((TOKENS, 1), jnp.int32),
            jax.ShapeDtypeStruct((TOKENS, 1), jnp.float32),
        ],
        scratch_shapes=[pltpu.VMEM((TOKENS, 1), jnp.float32)],
    )(logits, exponential, t)

    lane_best = jnp.max(sc_max, axis=1)
    lane_arg = jnp.min(
        jnp.where(sc_max == lane_best[:, None], sc_idx, VOCAB), axis=1)
    better = lane_best > tc_max[:, 0]
    return jnp.where(better, lane_arg, tc_arg[:, 0]).astype(jnp.int32)
